# baseline (speedup 1.0000x reference)
import functools

import jax
import jax.numpy as jnp
from jax import lax
from jax.experimental import pallas as pl
from jax.experimental.pallas import tpu as pltpu
from jax.experimental.pallas import tpu_sc as plsc

_NC, _NS = 2, 16
_NW = _NC * _NS
_BW = 128
_L = 16
_NG = _BW // _L


def _make(batch, seq, d, dpad):
    th_n, hi_n = d // 8, 8
    tb_n = batch // _BW

    mesh = plsc.VectorSubcoreMesh(
        core_axis_name="c", subcore_axis_name="s",
        num_cores=_NC, num_subcores=_NS)

    @functools.partial(
        pl.kernel,
        mesh=mesh,
        compiler_params=pltpu.CompilerParams(
            use_tc_tiling_on_sc=False, needs_layout_passes=False),
        out_type=jax.ShapeDtypeStruct((seq, th_n, tb_n, hi_n, _BW),
                                      jnp.float32),
        scratch_types=[
            pltpu.VMEM((_BW,), jnp.int32),
            pltpu.VMEM((_BW,), jnp.int32),
            pltpu.VMEM((_BW,), jnp.int32),
            pltpu.VMEM((_BW,), jnp.int32),
            pltpu.VMEM((_BW,), jnp.int32),
            pltpu.VMEM((_BW,), jnp.int32),
            pltpu.VMEM((2, _BW, dpad), jnp.float32),
            pltpu.VMEM((2, th_n, hi_n, _BW), jnp.float32),
            pltpu.SemaphoreType.DMA,
            pltpu.SemaphoreType.DMA,
            pltpu.SemaphoreType.DMA,
            pltpu.SemaphoreType.DMA,
            pltpu.SemaphoreType.DMA,
            pltpu.SemaphoreType.DMA,
        ],
    )
    def gather_kernel(idx_hbm, table_hbm, out_hbm,
                      pidx0, pidx1, sidx0, sidx1, par0, par1,
                      raw_v, slab_v,
                      psem0, psem1, gsem0, gsem1, osem0, osem1):
        wid = lax.axis_index("s") * _NC + lax.axis_index("c")
        wb = wid * _BW
        pidx = (pidx0, pidx1)
        sidx = (sidx0, sidx1)
        par = (par0, par1)
        psem = (psem0, psem1)
        gsem = (gsem0, gsem1)
        osem = (osem0, osem1)

        def fire_pidx(s, a):
            pltpu.async_copy(
                idx_hbm.at[pl.ds(s * batch + wb, _BW)], pidx[a], psem[a])

        def wait_pidx(a):
            pltpu.make_async_copy(
                idx_hbm.at[pl.ds(0, _BW)], pidx[a], psem[a]).wait()

        def prep(a):
            for j in range(_NG):
                v = pidx[a][pl.ds(j * _L, _L)]
                sidx[a][pl.ds(j * _L, _L)] = lax.shift_right_logical(v, 1)
                par[a][pl.ds(j * _L, _L)] = lax.shift_left(
                    lax.bitwise_and(v, 1), 6)

        def fire_gather(a):
            pltpu.async_copy(table_hbm.at[sidx[a]], raw_v.at[a], gsem[a])

        def wait_gather(a):
            pltpu.make_async_copy(
                table_hbm.at[pl.ds(0, _BW)], raw_v.at[a], gsem[a]).wait()

        def fire_out(s, a):
            for th in range(th_n):
                pltpu.async_copy(
                    slab_v.at[a, th], out_hbm.at[s, th, wid], osem[a])

        def wait_out(a):
            for th in range(th_n):
                pltpu.make_async_copy(
                    slab_v.at[a, th], out_hbm.at[0, th, 0], osem[a]).wait()

        lanes = lax.iota(jnp.int32, _L)
        bidx = [lanes + bg * _L for bg in range(_NG)]

        def transpose(a):
            pv = [par[a][pl.ds(bg * _L, _L)] for bg in range(_NG)]

            def h0body(h0, col):
                th_v = lax.shift_right_logical(col, 3)
                hi_v = lax.bitwise_and(col, 7)
                for bg in range(_NG):
                    v = plsc.load_gather(
                        raw_v.at[a], [bidx[bg], col + pv[bg]])
                    plsc.store_scatter(
                        slab_v.at[a], [th_v, hi_v, bidx[bg]], v)
                return lax.bitwise_and(col + 1, d - 1)

            lax.fori_loop(0, d, h0body, lanes)

        def step(s, a, fire_g=True, fire_p=True, wait_o=True):
            b = 1 - a
            if fire_g:
                wait_pidx(b)
                prep(b)
                fire_gather(b)
            wait_gather(a)
            if fire_p:
                fire_pidx(s + 2, a)
            if wait_o:
                wait_out(a)
            transpose(a)
            fire_out(s, a)

        fire_pidx(0, 0)
        fire_pidx(1, 1)
        wait_pidx(0)
        prep(0)
        fire_gather(0)
        step(0, 0, wait_o=False)
        step(1, 1, wait_o=False)

        def body(i, carry):
            step(2 * i + 2, 0)
            step(2 * i + 3, 1)
            return carry

        lax.fori_loop(0, (seq - 4) // 2, body, 0)

        step(seq - 2, 0, fire_p=False)
        step(seq - 1, 1, fire_g=False, fire_p=False)
        wait_out(0)
        wait_out(1)

    return gather_kernel


def kernel(input_ids, word_embeddings):
    batch, seq = input_ids.shape
    v, d = word_embeddings.shape
    dpad = 2 * d
    idx_flat = input_ids.T.reshape(-1).astype(jnp.int32)
    table2 = word_embeddings.reshape(v // 2, dpad)
    out5 = _make(batch, seq, d, dpad)(idx_flat, table2)
    out = out5.transpose(2, 4, 0, 1, 3).reshape(batch, seq, d)
    return out


# speedup vs baseline: 1.0937x; 1.0937x over previous
import functools

import jax
import jax.numpy as jnp
from jax import lax
from jax.experimental import pallas as pl
from jax.experimental.pallas import tpu as pltpu
from jax.experimental.pallas import tpu_sc as plsc

_NC, _NS = 2, 16
_NW = _NC * _NS
_BW = 128
_L = 16
_NG = _BW // _L


def _make(batch, seq, d, dpad):
    th_n, hi_n = d // 8, 8
    tb_n = batch // _BW

    mesh = plsc.VectorSubcoreMesh(
        core_axis_name="c", subcore_axis_name="s",
        num_cores=_NC, num_subcores=_NS)

    @functools.partial(
        pl.kernel,
        mesh=mesh,
        compiler_params=pltpu.CompilerParams(
            use_tc_tiling_on_sc=False, needs_layout_passes=False),
        out_type=jax.ShapeDtypeStruct((seq, th_n, tb_n, hi_n, _BW),
                                      jnp.float32),
        scratch_types=[
            pltpu.VMEM((_BW,), jnp.int32),
            pltpu.VMEM((_BW,), jnp.int32),
            pltpu.VMEM((_BW,), jnp.int32),
            pltpu.VMEM((_BW,), jnp.int32),
            pltpu.VMEM((_BW,), jnp.int32),
            pltpu.VMEM((_BW,), jnp.int32),
            pltpu.VMEM((2, _BW, dpad), jnp.float32),
            pltpu.VMEM((2, th_n, hi_n, _BW), jnp.float32),
            pltpu.SemaphoreType.DMA,
            pltpu.SemaphoreType.DMA,
            pltpu.SemaphoreType.DMA,
            pltpu.SemaphoreType.DMA,
            pltpu.SemaphoreType.DMA,
            pltpu.SemaphoreType.DMA,
        ],
    )
    def gather_kernel(idx_hbm, table_hbm, out_hbm,
                      pidx0, pidx1, sidx0, sidx1, par0, par1,
                      raw_v, slab_v,
                      psem0, psem1, gsem0, gsem1, osem0, osem1):
        wid = lax.axis_index("s") * _NC + lax.axis_index("c")
        wb = wid * _BW
        pidx = (pidx0, pidx1)
        sidx = (sidx0, sidx1)
        par = (par0, par1)
        psem = (psem0, psem1)
        gsem = (gsem0, gsem1)
        osem = (osem0, osem1)

        def fire_pidx(s, a):
            pltpu.async_copy(
                idx_hbm.at[pl.ds(s * batch + wb, _BW)], pidx[a], psem[a])

        def wait_pidx(a):
            pltpu.make_async_copy(
                idx_hbm.at[pl.ds(0, _BW)], pidx[a], psem[a]).wait()

        def prep(a):
            for j in range(_NG):
                v = pidx[a][pl.ds(j * _L, _L)]
                sidx[a][pl.ds(j * _L, _L)] = lax.shift_right_logical(v, 1)
                par[a][pl.ds(j * _L, _L)] = lax.shift_left(
                    lax.bitwise_and(v, 1), 6)

        def fire_gather(a):
            pltpu.async_copy(table_hbm.at[sidx[a]], raw_v.at[a], gsem[a])

        def wait_gather(a):
            pltpu.make_async_copy(
                table_hbm.at[pl.ds(0, _BW)], raw_v.at[a], gsem[a]).wait()

        def fire_out(s, a):
            for th in range(th_n):
                pltpu.async_copy(
                    slab_v.at[a, th], out_hbm.at[s, th, wid], osem[a])

        def wait_out(a):
            for th in range(th_n):
                pltpu.make_async_copy(
                    slab_v.at[a, th], out_hbm.at[0, th, 0], osem[a]).wait()

        lanes = lax.iota(jnp.int32, _L)
        bidx = [lanes + bg * _L for bg in range(_NG)]

        def transpose(a):
            pv = [par[a][pl.ds(bg * _L, _L)] for bg in range(_NG)]

            def h0body(h0, col):
                th_v = lax.shift_right_logical(col, 3)
                hi_v = lax.bitwise_and(col, 7)
                for bg in range(_NG):
                    v = plsc.load_gather(
                        raw_v.at[a], [bidx[bg], col + pv[bg]])
                    plsc.store_scatter(
                        slab_v.at[a], [th_v, hi_v, bidx[bg]], v)
                return lax.bitwise_and(col + 1, d - 1)

            lax.fori_loop(0, d, h0body, lanes)

        def step(s, a, fire_g=True, fire_p=True, wait_o=True):
            b = 1 - a
            if fire_g:
                wait_pidx(b)
                prep(b)
                fire_gather(b)
            wait_gather(a)
            if fire_p:
                fire_pidx(s + 2, a)
            if wait_o:
                wait_out(a)
            transpose(a)
            fire_out(s, a)

        fire_pidx(0, 0)
        fire_pidx(1, 1)
        wait_pidx(0)
        prep(0)
        fire_gather(0)
        step(0, 0, wait_o=False)
        step(1, 1, wait_o=False)

        def body(i, carry):
            step(2 * i + 2, 0)
            step(2 * i + 3, 1)
            return carry

        lax.fori_loop(0, (seq - 4) // 2, body, 0)

        step(seq - 2, 0, fire_p=False)
        step(seq - 1, 1, fire_g=False, fire_p=False)
        wait_out(0)
        wait_out(1)

    return gather_kernel


def _make_table(v, d, dpad):
    nb = v // 128
    per_w = nb // _NW
    extra = nb - per_w * _NW

    mesh = plsc.VectorSubcoreMesh(
        core_axis_name="c", subcore_axis_name="s",
        num_cores=_NC, num_subcores=_NS)

    @functools.partial(
        pl.kernel,
        mesh=mesh,
        compiler_params=pltpu.CompilerParams(
            use_tc_tiling_on_sc=True, needs_layout_passes=False),
        out_type=jax.ShapeDtypeStruct((v // 2, dpad), jnp.float32),
        scratch_types=[
            pltpu.VMEM((2, 64, 128), jnp.float32),
            pltpu.VMEM((2, 64, 128), jnp.float32),
            pltpu.VMEM((32, 128), jnp.float32),
            pltpu.SemaphoreType.DMA,
            pltpu.SemaphoreType.DMA,
            pltpu.SemaphoreType.DMA,
            pltpu.SemaphoreType.DMA,
            pltpu.SemaphoreType.DMA,
        ],
    )
    def tr_kernel(wt_hbm, tail_hbm, out_hbm, vin, vout, tl_v,
                  isem0, isem1, osem0, osem1, tsem):
        wid = lax.axis_index("s") * _NC + lax.axis_index("c")
        base = wid * per_w + jnp.minimum(wid, extra)
        isem = (isem0, isem1)
        osem = (osem0, osem1)

        def fire_in(blk, a):
            pltpu.async_copy(
                wt_hbm.at[:, pl.ds((base + blk) * 128, 128)], vin.at[a],
                isem[a])

        def wait_in(a):
            pltpu.make_async_copy(
                wt_hbm.at[:, pl.ds(0, 128)], vin.at[a], isem[a]).wait()

        def fire_out(blk, a):
            pltpu.async_copy(
                vout.at[a], out_hbm.at[pl.ds((base + blk) * 64, 64)],
                osem[a])

        def wait_out(a):
            pltpu.make_async_copy(
                vout.at[a], out_hbm.at[pl.ds(0, 64)], osem[a]).wait()

        lanes = lax.iota(jnp.int32, _L)
        jidx = [lanes + jg * _L for jg in range(8)]
        qp = [lax.shift_right_logical(j, 1) for j in jidx]
        cb = [lax.shift_left(lax.bitwise_and(j, 1), 6) for j in jidx]

        def transpose(a):
            def h0body(h0, col):
                for jg in range(8):
                    val = plsc.load_gather(vin.at[a], [col, jidx[jg]])
                    plsc.store_scatter(
                        vout.at[a], [qp[jg], cb[jg] + col], val)
                return lax.bitwise_and(col + 1, d - 1)

            lax.fori_loop(0, d, h0body, lanes)

        def step(blk, a, fire_nxt=True, wait_o=True):
            b = 1 - a
            if fire_nxt:
                pl.when(blk + 1 < per_w + (wid < extra))(
                    lambda: fire_in(blk + 1, b))
            wait_in(a)
            if wait_o:
                wait_out(a)
            transpose(a)
            fire_out(blk, a)

        @pl.when(wid == 0)
        def _():
            pltpu.async_copy(tail_hbm, tl_v, tsem)
            pltpu.make_async_copy(tail_hbm, tl_v, tsem).wait()
            pltpu.async_copy(
                tl_v, out_hbm.at[pl.ds((v // 128) * 64, (v % 128) // 2)],
                tsem)
            pltpu.make_async_copy(
                tl_v, out_hbm.at[pl.ds(0, (v % 128) // 2)], tsem).wait()

        fire_in(0, 0)
        step(0, 0, wait_o=False)
        step(1, 1, wait_o=False)

        def body(i, carry):
            step(2 * i + 2, 0)
            step(2 * i + 3, 1)
            return carry

        lax.fori_loop(0, (per_w - 4) // 2, body, 0)

        step(per_w - 2, 0)
        step(per_w - 1, 1)

        @pl.when(wid < extra)
        def _():
            wait_in(0)
            wait_out(0)
            transpose(0)
            fire_out(per_w, 0)
            wait_out(0)
        pl.when(wid >= extra)(lambda: wait_out(0))
        wait_out(1)

    return tr_kernel


def kernel(input_ids, word_embeddings):
    batch, seq = input_ids.shape
    v, d = word_embeddings.shape
    dpad = 2 * d
    idx_flat = input_ids.T.reshape(-1).astype(jnp.int32)
    wt = word_embeddings.T
    tail2 = word_embeddings[(v // 128) * 128:].reshape(-1, dpad)
    table2 = _make_table(v, d, dpad)(wt, tail2)
    out5 = _make(batch, seq, d, dpad)(idx_flat, table2)
    out = out5.transpose(2, 4, 0, 1, 3).reshape(batch, seq, d)
    return out


# speedup vs baseline: 1.0941x; 1.0004x over previous
import functools

import jax
import jax.numpy as jnp
from jax import lax
from jax.experimental import pallas as pl
from jax.experimental.pallas import tpu as pltpu
from jax.experimental.pallas import tpu_sc as plsc

_NC, _NS = 2, 16
_NW = _NC * _NS
_BW = 128
_L = 16
_NG = _BW // _L


def _make(batch, seq, d, dpad):
    th_n, hi_n = d // 8, 8
    tb_n = batch // _BW

    mesh = plsc.VectorSubcoreMesh(
        core_axis_name="c", subcore_axis_name="s",
        num_cores=_NC, num_subcores=_NS)

    @functools.partial(
        pl.kernel,
        mesh=mesh,
        compiler_params=pltpu.CompilerParams(
            use_tc_tiling_on_sc=False, needs_layout_passes=False),
        out_type=jax.ShapeDtypeStruct((seq, th_n, tb_n, hi_n, _BW),
                                      jnp.float32),
        scratch_types=[
            pltpu.VMEM((_BW,), jnp.int32),
            pltpu.VMEM((_BW,), jnp.int32),
            pltpu.VMEM((_BW,), jnp.int32),
            pltpu.VMEM((_BW,), jnp.int32),
            pltpu.VMEM((_BW,), jnp.int32),
            pltpu.VMEM((_BW,), jnp.int32),
            pltpu.VMEM((2, _BW, dpad), jnp.float32),
            pltpu.VMEM((2, d, _BW), jnp.float32),
            pltpu.SemaphoreType.DMA,
            pltpu.SemaphoreType.DMA,
            pltpu.SemaphoreType.DMA,
            pltpu.SemaphoreType.DMA,
            pltpu.SemaphoreType.DMA,
            pltpu.SemaphoreType.DMA,
        ],
    )
    def gather_kernel(idx_hbm, table_hbm, out_hbm,
                      pidx0, pidx1, sidx0, sidx1, par0, par1,
                      raw_v, slab_v,
                      psem0, psem1, gsem0, gsem1, osem0, osem1):
        wid = lax.axis_index("s") * _NC + lax.axis_index("c")
        wb = wid * _BW
        pidx = (pidx0, pidx1)
        sidx = (sidx0, sidx1)
        par = (par0, par1)
        psem = (psem0, psem1)
        gsem = (gsem0, gsem1)
        osem = (osem0, osem1)

        def fire_pidx(s, a):
            pltpu.async_copy(
                idx_hbm.at[pl.ds(s * batch + wb, _BW)], pidx[a], psem[a])

        def wait_pidx(a):
            pltpu.make_async_copy(
                idx_hbm.at[pl.ds(0, _BW)], pidx[a], psem[a]).wait()

        def prep(a):
            for j in range(_NG):
                v = pidx[a][pl.ds(j * _L, _L)]
                sidx[a][pl.ds(j * _L, _L)] = lax.shift_right_logical(v, 1)
                par[a][pl.ds(j * _L, _L)] = lax.shift_left(
                    lax.bitwise_and(v, 1), 6)

        def fire_gather(a):
            pltpu.async_copy(table_hbm.at[sidx[a]], raw_v.at[a], gsem[a])

        def wait_gather(a):
            pltpu.make_async_copy(
                table_hbm.at[pl.ds(0, _BW)], raw_v.at[a], gsem[a]).wait()

        def fire_out(s, a):
            for th in range(th_n):
                pltpu.async_copy(
                    slab_v.at[a].at[pl.ds(th * hi_n, hi_n)],
                    out_hbm.at[s, th, wid], osem[a])

        def wait_out(a):
            for th in range(th_n):
                pltpu.make_async_copy(
                    slab_v.at[a].at[pl.ds(th * hi_n, hi_n)],
                    out_hbm.at[0, th, 0], osem[a]).wait()

        lanes = lax.iota(jnp.int32, _L)
        bidx = [lanes + bg * _L for bg in range(_NG)]

        def transpose(a):
            pv = [par[a][pl.ds(bg * _L, _L)] for bg in range(_NG)]

            def h0body(i, col):
                for _ in range(2):
                    for bg in range(_NG):
                        v = plsc.load_gather(
                            raw_v.at[a], [bidx[bg], col + pv[bg]])
                        plsc.store_scatter(
                            slab_v.at[a], [col, bidx[bg]], v)
                    col = lax.bitwise_and(col + 1, d - 1)
                return col

            lax.fori_loop(0, d // 2, h0body, lanes)

        def step(s, a, fire_g=True, fire_p=True, wait_o=True):
            b = 1 - a
            if fire_g:
                wait_pidx(b)
                prep(b)
                fire_gather(b)
            wait_gather(a)
            if fire_p:
                fire_pidx(s + 2, a)
            if wait_o:
                wait_out(a)
            transpose(a)
            fire_out(s, a)

        fire_pidx(0, 0)
        fire_pidx(1, 1)
        wait_pidx(0)
        prep(0)
        fire_gather(0)
        step(0, 0, wait_o=False)
        step(1, 1, wait_o=False)

        def body(i, carry):
            step(2 * i + 2, 0)
            step(2 * i + 3, 1)
            return carry

        lax.fori_loop(0, (seq - 4) // 2, body, 0)

        step(seq - 2, 0, fire_p=False)
        step(seq - 1, 1, fire_g=False, fire_p=False)
        wait_out(0)
        wait_out(1)

    return gather_kernel


def _make_table(v, d, dpad):
    nb = v // 128
    per_w = nb // _NW
    extra = nb - per_w * _NW

    mesh = plsc.VectorSubcoreMesh(
        core_axis_name="c", subcore_axis_name="s",
        num_cores=_NC, num_subcores=_NS)

    @functools.partial(
        pl.kernel,
        mesh=mesh,
        compiler_params=pltpu.CompilerParams(
            use_tc_tiling_on_sc=True, needs_layout_passes=False),
        out_type=jax.ShapeDtypeStruct((v // 2, dpad), jnp.float32),
        scratch_types=[
            pltpu.VMEM((2, 64, 128), jnp.float32),
            pltpu.VMEM((2, 64, 128), jnp.float32),
            pltpu.VMEM((32, 128), jnp.float32),
            pltpu.SemaphoreType.DMA,
            pltpu.SemaphoreType.DMA,
            pltpu.SemaphoreType.DMA,
            pltpu.SemaphoreType.DMA,
            pltpu.SemaphoreType.DMA,
        ],
    )
    def tr_kernel(wt_hbm, tail_hbm, out_hbm, vin, vout, tl_v,
                  isem0, isem1, osem0, osem1, tsem):
        wid = lax.axis_index("s") * _NC + lax.axis_index("c")
        base = wid * per_w + jnp.minimum(wid, extra)
        isem = (isem0, isem1)
        osem = (osem0, osem1)

        def fire_in(blk, a):
            pltpu.async_copy(
                wt_hbm.at[:, pl.ds((base + blk) * 128, 128)], vin.at[a],
                isem[a])

        def wait_in(a):
            pltpu.make_async_copy(
                wt_hbm.at[:, pl.ds(0, 128)], vin.at[a], isem[a]).wait()

        def fire_out(blk, a):
            pltpu.async_copy(
                vout.at[a], out_hbm.at[pl.ds((base + blk) * 64, 64)],
                osem[a])

        def wait_out(a):
            pltpu.make_async_copy(
                vout.at[a], out_hbm.at[pl.ds(0, 64)], osem[a]).wait()

        lanes = lax.iota(jnp.int32, _L)
        jidx = [lanes + jg * _L for jg in range(8)]
        qp = [lax.shift_right_logical(j, 1) for j in jidx]
        cb = [lax.shift_left(lax.bitwise_and(j, 1), 6) for j in jidx]

        def transpose(a):
            def h0body(i, col):
                for _ in range(2):
                    for jg in range(8):
                        val = plsc.load_gather(vin.at[a], [col, jidx[jg]])
                        plsc.store_scatter(
                            vout.at[a], [qp[jg], cb[jg] + col], val)
                    col = lax.bitwise_and(col + 1, d - 1)
                return col

            lax.fori_loop(0, d // 2, h0body, lanes)

        def step(blk, a, fire_nxt=True, wait_o=True):
            b = 1 - a
            if fire_nxt:
                pl.when(blk + 1 < per_w + (wid < extra))(
                    lambda: fire_in(blk + 1, b))
            wait_in(a)
            if wait_o:
                wait_out(a)
            transpose(a)
            fire_out(blk, a)

        @pl.when(wid == 0)
        def _():
            pltpu.async_copy(tail_hbm, tl_v, tsem)
            pltpu.make_async_copy(tail_hbm, tl_v, tsem).wait()
            pltpu.async_copy(
                tl_v, out_hbm.at[pl.ds((v // 128) * 64, (v % 128) // 2)],
                tsem)
            pltpu.make_async_copy(
                tl_v, out_hbm.at[pl.ds(0, (v % 128) // 2)], tsem).wait()

        fire_in(0, 0)
        step(0, 0, wait_o=False)
        step(1, 1, wait_o=False)

        def body(i, carry):
            step(2 * i + 2, 0)
            step(2 * i + 3, 1)
            return carry

        lax.fori_loop(0, (per_w - 4) // 2, body, 0)

        step(per_w - 2, 0)
        step(per_w - 1, 1)

        @pl.when(wid < extra)
        def _():
            wait_in(0)
            wait_out(0)
            transpose(0)
            fire_out(per_w, 0)
            wait_out(0)
        pl.when(wid >= extra)(lambda: wait_out(0))
        wait_out(1)

    return tr_kernel


def kernel(input_ids, word_embeddings):
    batch, seq = input_ids.shape
    v, d = word_embeddings.shape
    dpad = 2 * d
    idx_flat = input_ids.T.reshape(-1).astype(jnp.int32)
    wt = word_embeddings.T
    tail2 = word_embeddings[(v // 128) * 128:].reshape(-1, dpad)
    table2 = _make_table(v, d, dpad)(wt, tail2)
    out5 = _make(batch, seq, d, dpad)(idx_flat, table2)
    out = out5.transpose(2, 4, 0, 1, 3).reshape(batch, seq, d)
    return out


# speedup vs baseline: 1.9933x; 1.8218x over previous
import functools

import jax
import jax.numpy as jnp
from jax import lax
from jax.experimental import pallas as pl
from jax.experimental.pallas import tpu as pltpu
from jax.experimental.pallas import tpu_sc as plsc

_NC, _NS = 2, 16
_NW = _NC * _NS
_BW = 128
_L = 16
_NG = _BW // _L


def _make(batch, seq, d, dpad):
    th_n, hi_n = d // 8, 8
    tb_n = batch // _BW

    mesh = plsc.VectorSubcoreMesh(
        core_axis_name="c", subcore_axis_name="s",
        num_cores=_NC, num_subcores=_NS)

    @functools.partial(
        pl.kernel,
        mesh=mesh,
        compiler_params=pltpu.CompilerParams(
            use_tc_tiling_on_sc=False, needs_layout_passes=False),
        out_type=jax.ShapeDtypeStruct((seq, th_n, tb_n, hi_n, _BW),
                                      jnp.float32),
        scratch_types=[
            pltpu.VMEM((_BW,), jnp.int32),
            pltpu.VMEM((_BW,), jnp.int32),
            pltpu.VMEM((_BW,), jnp.int32),
            pltpu.VMEM((_BW,), jnp.int32),
            pltpu.VMEM((_BW,), jnp.int32),
            pltpu.VMEM((_BW,), jnp.int32),
            pltpu.VMEM((2, _BW, dpad), jnp.float32),
            pltpu.VMEM((2, d, _BW), jnp.float32),
            pltpu.SemaphoreType.DMA,
            pltpu.SemaphoreType.DMA,
            pltpu.SemaphoreType.DMA,
            pltpu.SemaphoreType.DMA,
            pltpu.SemaphoreType.DMA,
            pltpu.SemaphoreType.DMA,
        ],
    )
    def gather_kernel(idx_hbm, table_hbm, out_hbm,
                      pidx0, pidx1, sidx0, sidx1, par0, par1,
                      raw_v, slab_v,
                      psem0, psem1, gsem0, gsem1, osem0, osem1):
        wid = lax.axis_index("s") * _NC + lax.axis_index("c")
        wb = wid * _BW
        pidx = (pidx0, pidx1)
        sidx = (sidx0, sidx1)
        par = (par0, par1)
        psem = (psem0, psem1)
        gsem = (gsem0, gsem1)
        osem = (osem0, osem1)

        def fire_pidx(s, a):
            pltpu.async_copy(
                idx_hbm.at[pl.ds(s * batch + wb, _BW)], pidx[a], psem[a])

        def wait_pidx(a):
            pltpu.make_async_copy(
                idx_hbm.at[pl.ds(0, _BW)], pidx[a], psem[a]).wait()

        def prep(a):
            for j in range(_NG):
                v = pidx[a][pl.ds(j * _L, _L)]
                sidx[a][pl.ds(j * _L, _L)] = lax.shift_right_logical(v, 1)
                par[a][pl.ds(j * _L, _L)] = lax.shift_left(
                    lax.bitwise_and(v, 1), 6)

        def fire_gather(a):
            pltpu.async_copy(table_hbm.at[sidx[a]], raw_v.at[a], gsem[a])

        def wait_gather(a):
            pltpu.make_async_copy(
                table_hbm.at[pl.ds(0, _BW)], raw_v.at[a], gsem[a]).wait()

        def fire_out(s, a):
            for th in range(th_n):
                pltpu.async_copy(
                    slab_v.at[a].at[pl.ds(th * hi_n, hi_n)],
                    out_hbm.at[s, th, wid], osem[a])

        def wait_out(a):
            for th in range(th_n):
                pltpu.make_async_copy(
                    slab_v.at[a].at[pl.ds(th * hi_n, hi_n)],
                    out_hbm.at[0, th, 0], osem[a]).wait()

        lanes = lax.iota(jnp.int32, _L)
        bidx = [lanes + bg * _L for bg in range(_NG)]

        def transpose(a):
            pv = [par[a][pl.ds(bg * _L, _L)] for bg in range(_NG)]

            def h0body(i, col):
                for _ in range(2):
                    vals = [plsc.load_gather(
                        raw_v.at[a], [bidx[bg], col + pv[bg]])
                        for bg in range(_NG)]
                    for bg in range(_NG):
                        plsc.store_scatter(
                            slab_v.at[a], [col, bidx[bg]], vals[bg])
                    col = lax.bitwise_and(col + 1, d - 1)
                return col

            lax.fori_loop(0, d // 2, h0body, lanes)

        def step(s, a, fire_g=True, fire_p=True, wait_o=True):
            b = 1 - a
            if fire_g:
                wait_pidx(b)
                prep(b)
                fire_gather(b)
            wait_gather(a)
            if fire_p:
                fire_pidx(s + 2, a)
            if wait_o:
                wait_out(a)
            transpose(a)
            fire_out(s, a)

        fire_pidx(0, 0)
        fire_pidx(1, 1)
        wait_pidx(0)
        prep(0)
        fire_gather(0)
        step(0, 0, wait_o=False)
        step(1, 1, wait_o=False)

        def body(i, carry):
            step(2 * i + 2, 0)
            step(2 * i + 3, 1)
            return carry

        lax.fori_loop(0, (seq - 4) // 2, body, 0)

        step(seq - 2, 0, fire_p=False)
        step(seq - 1, 1, fire_g=False, fire_p=False)
        wait_out(0)
        wait_out(1)

    return gather_kernel


def _make_table(v, d, dpad):
    nb = v // 128
    per_w = nb // _NW
    extra = nb - per_w * _NW

    mesh = plsc.VectorSubcoreMesh(
        core_axis_name="c", subcore_axis_name="s",
        num_cores=_NC, num_subcores=_NS)

    @functools.partial(
        pl.kernel,
        mesh=mesh,
        compiler_params=pltpu.CompilerParams(
            use_tc_tiling_on_sc=True, needs_layout_passes=False),
        out_type=jax.ShapeDtypeStruct((v // 2, dpad), jnp.float32),
        scratch_types=[
            pltpu.VMEM((2, 64, 128), jnp.float32),
            pltpu.VMEM((2, 64, 128), jnp.float32),
            pltpu.VMEM((32, 128), jnp.float32),
            pltpu.SemaphoreType.DMA,
            pltpu.SemaphoreType.DMA,
            pltpu.SemaphoreType.DMA,
            pltpu.SemaphoreType.DMA,
            pltpu.SemaphoreType.DMA,
        ],
    )
    def tr_kernel(wt_hbm, tail_hbm, out_hbm, vin, vout, tl_v,
                  isem0, isem1, osem0, osem1, tsem):
        wid = lax.axis_index("s") * _NC + lax.axis_index("c")
        base = wid * per_w + jnp.minimum(wid, extra)
        isem = (isem0, isem1)
        osem = (osem0, osem1)

        def fire_in(blk, a):
            pltpu.async_copy(
                wt_hbm.at[:, pl.ds((base + blk) * 128, 128)], vin.at[a],
                isem[a])

        def wait_in(a):
            pltpu.make_async_copy(
                wt_hbm.at[:, pl.ds(0, 128)], vin.at[a], isem[a]).wait()

        def fire_out(blk, a):
            pltpu.async_copy(
                vout.at[a], out_hbm.at[pl.ds((base + blk) * 64, 64)],
                osem[a])

        def wait_out(a):
            pltpu.make_async_copy(
                vout.at[a], out_hbm.at[pl.ds(0, 64)], osem[a]).wait()

        lanes = lax.iota(jnp.int32, _L)
        jidx = [lanes + jg * _L for jg in range(8)]
        qp = [lax.shift_right_logical(j, 1) for j in jidx]
        cb = [lax.shift_left(lax.bitwise_and(j, 1), 6) for j in jidx]

        def transpose(a):
            def h0body(i, col):
                for _ in range(2):
                    vals = [plsc.load_gather(vin.at[a], [col, jidx[jg]])
                            for jg in range(8)]
                    for jg in range(8):
                        plsc.store_scatter(
                            vout.at[a], [qp[jg], cb[jg] + col], vals[jg])
                    col = lax.bitwise_and(col + 1, d - 1)
                return col

            lax.fori_loop(0, d // 2, h0body, lanes)

        def step(blk, a, fire_nxt=True, wait_o=True):
            b = 1 - a
            if fire_nxt:
                pl.when(blk + 1 < per_w + (wid < extra))(
                    lambda: fire_in(blk + 1, b))
            wait_in(a)
            if wait_o:
                wait_out(a)
            transpose(a)
            fire_out(blk, a)

        @pl.when(wid == 0)
        def _():
            pltpu.async_copy(tail_hbm, tl_v, tsem)
            pltpu.make_async_copy(tail_hbm, tl_v, tsem).wait()
            pltpu.async_copy(
                tl_v, out_hbm.at[pl.ds((v // 128) * 64, (v % 128) // 2)],
                tsem)
            pltpu.make_async_copy(
                tl_v, out_hbm.at[pl.ds(0, (v % 128) // 2)], tsem).wait()

        fire_in(0, 0)
        step(0, 0, wait_o=False)
        step(1, 1, wait_o=False)

        def body(i, carry):
            step(2 * i + 2, 0)
            step(2 * i + 3, 1)
            return carry

        lax.fori_loop(0, (per_w - 4) // 2, body, 0)

        step(per_w - 2, 0)
        step(per_w - 1, 1)

        @pl.when(wid < extra)
        def _():
            wait_in(0)
            wait_out(0)
            transpose(0)
            fire_out(per_w, 0)
            wait_out(0)
        pl.when(wid >= extra)(lambda: wait_out(0))
        wait_out(1)

    return tr_kernel


def kernel(input_ids, word_embeddings):
    batch, seq = input_ids.shape
    v, d = word_embeddings.shape
    dpad = 2 * d
    idx_flat = input_ids.T.reshape(-1).astype(jnp.int32)
    wt = word_embeddings.T
    tail2 = word_embeddings[(v // 128) * 128:].reshape(-1, dpad)
    table2 = _make_table(v, d, dpad)(wt, tail2)
    out5 = _make(batch, seq, d, dpad)(idx_flat, table2)
    out = out5.transpose(2, 4, 0, 1, 3).reshape(batch, seq, d)
    return out


# speedup vs baseline: 2.2938x; 1.1508x over previous
import functools

import jax
import jax.numpy as jnp
from jax import lax
from jax.experimental import pallas as pl
from jax.experimental.pallas import tpu as pltpu
from jax.experimental.pallas import tpu_sc as plsc

_NC, _NS = 2, 16
_NW = _NC * _NS
_BW = 128
_L = 16
_NG = _BW // _L


def _make(batch, seq, d):
    th_n, hi_n = d // 8, 8
    tb_n = batch // _BW

    mesh = plsc.VectorSubcoreMesh(
        core_axis_name="c", subcore_axis_name="s",
        num_cores=_NC, num_subcores=_NS)

    @functools.partial(
        pl.kernel,
        mesh=mesh,
        compiler_params=pltpu.CompilerParams(
            use_tc_tiling_on_sc=False, needs_layout_passes=False),
        out_type=jax.ShapeDtypeStruct((seq, th_n, tb_n, hi_n, _BW),
                                      jnp.float32),
        scratch_types=[
            pltpu.VMEM((_BW,), jnp.int32),
            pltpu.VMEM((_BW,), jnp.int32),
            pltpu.VMEM((2, _BW, d), jnp.float32),
            pltpu.VMEM((2, d, _BW), jnp.float32),
            pltpu.SemaphoreType.DMA,
            pltpu.SemaphoreType.DMA,
            pltpu.SemaphoreType.DMA,
            pltpu.SemaphoreType.DMA,
            pltpu.SemaphoreType.DMA,
            pltpu.SemaphoreType.DMA,
        ],
    )
    def gather_kernel(idx_hbm, table_hbm, out_hbm,
                      pidx0, pidx1, raw_v, slab_v,
                      psem0, psem1, gsem0, gsem1, osem0, osem1):
        wid = lax.axis_index("s") * _NC + lax.axis_index("c")
        wb = wid * _BW
        pidx = (pidx0, pidx1)
        psem = (psem0, psem1)
        gsem = (gsem0, gsem1)
        osem = (osem0, osem1)

        def fire_pidx(s, a):
            pltpu.async_copy(
                idx_hbm.at[pl.ds(s * batch + wb, _BW)], pidx[a], psem[a])

        def wait_pidx(a):
            pltpu.make_async_copy(
                idx_hbm.at[pl.ds(0, _BW)], pidx[a], psem[a]).wait()

        def fire_gather(a):
            pltpu.async_copy(table_hbm.at[pidx[a]], raw_v.at[a], gsem[a])

        def wait_gather(a):
            pltpu.make_async_copy(
                table_hbm.at[pl.ds(0, _BW)], raw_v.at[a], gsem[a]).wait()

        def fire_out(s, a):
            for th in range(th_n):
                pltpu.async_copy(
                    slab_v.at[a].at[pl.ds(th * hi_n, hi_n)],
                    out_hbm.at[s, th, wid], osem[a])

        def wait_out(a):
            for th in range(th_n):
                pltpu.make_async_copy(
                    slab_v.at[a].at[pl.ds(th * hi_n, hi_n)],
                    out_hbm.at[0, th, 0], osem[a]).wait()

        lanes = lax.iota(jnp.int32, _L)
        bidx = [lanes + bg * _L for bg in range(_NG)]

        def transpose(a):
            def h0body(i, col):
                for _ in range(2):
                    vals = [plsc.load_gather(
                        raw_v.at[a], [bidx[bg], col])
                        for bg in range(_NG)]
                    for bg in range(_NG):
                        plsc.store_scatter(
                            slab_v.at[a], [col, bidx[bg]], vals[bg])
                    col = lax.bitwise_and(col + 1, d - 1)
                return col

            lax.fori_loop(0, d // 2, h0body, lanes)

        def step(s, a, fire_g=True, fire_p=True, wait_o=True):
            b = 1 - a
            if fire_g:
                wait_pidx(b)
                fire_gather(b)
            wait_gather(a)
            if fire_p:
                fire_pidx(s + 2, a)
            if wait_o:
                wait_out(a)
            transpose(a)
            fire_out(s, a)

        fire_pidx(0, 0)
        fire_pidx(1, 1)
        wait_pidx(0)
        fire_gather(0)
        step(0, 0, wait_o=False)
        step(1, 1, wait_o=False)

        def body(i, carry):
            step(2 * i + 2, 0)
            step(2 * i + 3, 1)
            return carry

        lax.fori_loop(0, (seq - 4) // 2, body, 0)

        step(seq - 2, 0, fire_p=False)
        step(seq - 1, 1, fire_g=False, fire_p=False)
        wait_out(0)
        wait_out(1)

    return gather_kernel


def _make_table(v, d):
    nb = v // 128
    per_w = nb // _NW
    extra = nb - per_w * _NW

    mesh = plsc.VectorSubcoreMesh(
        core_axis_name="c", subcore_axis_name="s",
        num_cores=_NC, num_subcores=_NS)

    @functools.partial(
        pl.kernel,
        mesh=mesh,
        compiler_params=pltpu.CompilerParams(
            use_tc_tiling_on_sc=True, needs_layout_passes=False),
        out_type=jax.ShapeDtypeStruct((v * d,), jnp.float32),
        scratch_types=[
            pltpu.VMEM((2, 64, 128), jnp.float32),
            pltpu.VMEM((64 * 128,), jnp.float32),
            pltpu.VMEM((64 * 128,), jnp.float32),
            pltpu.VMEM((4096,), jnp.float32),
            pltpu.SemaphoreType.DMA,
            pltpu.SemaphoreType.DMA,
            pltpu.SemaphoreType.DMA,
            pltpu.SemaphoreType.DMA,
            pltpu.SemaphoreType.DMA,
        ],
    )
    def tr_kernel(wt_hbm, tail_hbm, out_hbm, vin, vout0, vout1, tl_v,
                  isem0, isem1, osem0, osem1, tsem):
        wid = lax.axis_index("s") * _NC + lax.axis_index("c")
        base = wid * per_w + jnp.minimum(wid, extra)
        isem = (isem0, isem1)
        osem = (osem0, osem1)
        vout = (vout0, vout1)

        def fire_in(blk, a):
            pltpu.async_copy(
                wt_hbm.at[:, pl.ds((base + blk) * 128, 128)], vin.at[a],
                isem[a])

        def wait_in(a):
            pltpu.make_async_copy(
                wt_hbm.at[:, pl.ds(0, 128)], vin.at[a], isem[a]).wait()

        def fire_out(blk, a):
            pltpu.async_copy(
                vout[a],
                out_hbm.at[pl.ds((base + blk) * (64 * 128), 64 * 128)],
                osem[a])

        def wait_out(a):
            pltpu.make_async_copy(
                vout[a], out_hbm.at[pl.ds(0, 64 * 128)], osem[a]).wait()

        lanes = lax.iota(jnp.int32, _L)
        jidx = [lanes + jg * _L for jg in range(8)]
        j64 = [lax.shift_left(j, 6) for j in jidx]

        def transpose(a):
            def h0body(i, col):
                for _ in range(2):
                    vals = [plsc.load_gather(vin.at[a], [col, jidx[jg]])
                            for jg in range(8)]
                    for jg in range(8):
                        plsc.store_scatter(
                            vout[a], [j64[jg] + col], vals[jg])
                    col = lax.bitwise_and(col + 1, d - 1)
                return col

            lax.fori_loop(0, d // 2, h0body, lanes)

        def step(blk, a, fire_nxt=True, wait_o=True):
            b = 1 - a
            if fire_nxt:
                pl.when(blk + 1 < per_w + (wid < extra))(
                    lambda: fire_in(blk + 1, b))
            wait_in(a)
            if wait_o:
                wait_out(a)
            transpose(a)
            fire_out(blk, a)

        @pl.when(wid == 0)
        def _():
            pltpu.async_copy(tail_hbm, tl_v, tsem)
            pltpu.make_async_copy(tail_hbm, tl_v, tsem).wait()
            pltpu.async_copy(
                tl_v, out_hbm.at[pl.ds((v // 128) * 128 * d, (v % 128) * d)],
                tsem)
            pltpu.make_async_copy(
                tl_v, out_hbm.at[pl.ds(0, (v % 128) * d)], tsem).wait()

        fire_in(0, 0)
        step(0, 0, wait_o=False)
        step(1, 1, wait_o=False)

        def body(i, carry):
            step(2 * i + 2, 0)
            step(2 * i + 3, 1)
            return carry

        lax.fori_loop(0, (per_w - 4) // 2, body, 0)

        step(per_w - 2, 0)
        step(per_w - 1, 1)

        @pl.when(wid < extra)
        def _():
            wait_in(0)
            wait_out(0)
            transpose(0)
            fire_out(per_w, 0)
            wait_out(0)
        pl.when(wid >= extra)(lambda: wait_out(0))
        wait_out(1)

    return tr_kernel


def kernel(input_ids, word_embeddings):
    batch, seq = input_ids.shape
    v, d = word_embeddings.shape
    idx_flat = input_ids.T.reshape(-1).astype(jnp.int32)
    wt = word_embeddings.T
    tail2 = word_embeddings[(v // 128) * 128:].reshape(-1)
    table2 = _make_table(v, d)(wt, tail2).reshape(v, d)
    out5 = _make(batch, seq, d)(idx_flat, table2)
    out = out5.transpose(2, 4, 0, 1, 3).reshape(batch, seq, d)
    return out
